# baseline (speedup 1.0000x reference)
import functools

import jax
import jax.numpy as jnp
import numpy as np
from jax import lax
from jax.experimental import pallas as pl
from jax.experimental.pallas import tpu as pltpu
from jax.experimental.pallas import tpu_sc as plsc

_CLIP = 256.0
_EPS = float(np.finfo(np.float32).eps)
_LB = tuple(v / 256.0 for v in (0.0, 15.0, 30.0, 60.0, 96.0, 256.0))
_RB = tuple(v / 256.0 for v in (30.0, 60.0, 120.0, 240.0, 768.0, 768.0))
_N_TGT = 30
_L = 16
_PTILE = 8192
_NROW = 11


def _levels_to_bounds(lvlf):
    lb = jnp.full_like(lvlf, _LB[0])
    rb = jnp.full_like(lvlf, _RB[0])
    for i in range(1, 6):
        sel = lvlf > (i - 0.5)
        lb = jnp.where(sel, _LB[i], lb)
        rb = jnp.where(sel, _RB[i], rb)
    return lb, rb


def _sc_log(x):
    bits = lax.bitcast_convert_type(x, jnp.int32)
    e = jnp.bitwise_and(lax.shift_right_logical(bits, 23), 0xFF)
    ef = (e - 127).astype(jnp.float32)
    mbits = jnp.bitwise_or(jnp.bitwise_and(bits, 0x007FFFFF), 0x3F800000)
    m = lax.bitcast_convert_type(mbits, jnp.float32)
    big = m > 1.4142135381698608
    m = jnp.where(big, 0.5 * m, m)
    ef = jnp.where(big, ef + 1.0, ef)
    t = (m - 1.0) / (m + 1.0)
    t2 = t * t
    p = t * (2.0 + t2 * (0.6666666666666666
                         + t2 * (0.4 + t2 * 0.2857142857142857)))
    return ef * 0.6931471805599453 + p


def _iou_v(pl0, pl1, tl0, tl1):
    inter = jnp.minimum(pl0, tl0) + jnp.minimum(pl1, tl1)
    union = (tl0 + tl1) + (pl0 + pl1) - inter
    return inter / jnp.maximum(union, _EPS)


def _focal_v(a, b, is0):
    mx = jnp.maximum(a, b)
    za = jnp.exp(a - mx)
    zb = jnp.exp(b - mx)
    pt = jnp.where(is0, za, zb) / (za + zb)
    alpha_t = jnp.where(is0, 0.25, 0.75)
    omp = 1.0 - pt
    return -alpha_t * omp * omp * _sc_log(jnp.maximum(pt, _EPS))


def _lanes(vals):
    lane = jnp.arange(_L, dtype=jnp.int32)
    v = jnp.zeros((_L,), jnp.float32)
    for i, s in enumerate(vals):
        v = jnp.where(lane == i, s, v)
    return v


def _sc_body(x_hbm, t_hbm, out_hbm, xv, tv, scr, ov):
    ci = lax.axis_index("c")
    si = lax.axis_index("s")
    b = si

    @pl.when(ci == 0)
    def _():
        _sc_sample(x_hbm, t_hbm, out_hbm, xv, tv, scr, ov, b)


def _sc_sample(x_hbm, t_hbm, out_hbm, xv, tv, scr, ov, b):
    pltpu.sync_copy(x_hbm.at[b], xv)
    pltpu.sync_copy(t_hbm.at[b], tv)

    nchunk = _PTILE // _L
    P = _PTILE
    lane = jnp.arange(_L, dtype=jnp.int32)

    def pass1(i, acc):
        m_acc, np_acc, ll_acc, lc_acc, lct_acc = acc
        o = i * _L
        c = xv[pl.ds(0 * P + o, _L)]
        lvl = xv[pl.ds(1 * P + o, _L)]
        lb, rb = _levels_to_bounds(lvl)

        best = jnp.full((_L,), 2.0, jnp.float32)
        bs = jnp.zeros((_L,), jnp.float32)
        be = jnp.zeros((_L,), jnp.float32)
        for m in range(_N_TGT):
            s = tv[pl.ds(m * 2 * _L, _L)]
            e = tv[pl.ds(m * 2 * _L + _L, _L)]
            t1 = c - s
            t2 = e - c
            a = t1 + t2
            mn = jnp.minimum(t1, t2)
            mx = jnp.maximum(t1, t2)
            take = (mn >= 0.0) & (mx > lb) & (mx <= rb) & (a < best)
            best = jnp.where(take, a, best)
            bs = jnp.where(take, s, bs)
            be = jnp.where(take, e, be)

        found = best < 2.0
        posf = found.astype(jnp.float32)
        lt0 = (c - bs) * _CLIP
        lt1 = (be - c) * _CLIP
        ll = xv[pl.ds(2 * P + o, _L)]
        lr = xv[pl.ds(3 * P + o, _L)]
        iou = _iou_v(ll, lr, lt0, lt1)

        scr[pl.ds(0 * P + o, _L)] = iou
        scr[pl.ds(1 * P + o, _L)] = posf
        scr[pl.ds(2 * P + o, _L)] = lt0
        scr[pl.ds(3 * P + o, _L)] = lt1

        pred_area = ll + lr
        target_area = lt0 + lt1
        inter = jnp.minimum(ll, lt0) + jnp.minimum(lr, lt1)
        union = target_area + pred_area - inter
        ious = inter / jnp.maximum(union, _EPS)
        ac = jnp.maximum(ll, lt0) + jnp.maximum(lr, lt1)
        gious = ious - (ac - union) / jnp.maximum(ac, _EPS)

        p0 = xv[pl.ds(6 * P + o, _L)]
        p1 = xv[pl.ds(7 * P + o, _L)]
        ctr = xv[pl.ds(10 * P + o, _L)]
        half_w = 0.5 * pred_area
        cur0 = half_w * p0 + ll
        cur1 = half_w * p1 + lr
        ious_ct = jnp.maximum(_iou_v(cur0, cur1, lt0, lt1), 0.0)
        bce = (jnp.maximum(ctr, 0.0) - ctr * ious_ct
               + _sc_log(1.0 + jnp.exp(-jnp.abs(ctr))))

        g0 = xv[pl.ds(4 * P + o, _L)]
        g1 = xv[pl.ds(5 * P + o, _L)]
        fc = _focal_v(g0, g1, jnp.logical_not(found))

        m_acc = jnp.maximum(m_acc, jnp.where(found, iou, -3.0e38))
        np_acc = np_acc + posf
        ll_acc = ll_acc + (1.0 - gious) * posf
        lc_acc = lc_acc + fc
        lct_acc = lct_acc + bce * posf
        return m_acc, np_acc, ll_acc, lc_acc, lct_acc

    zeros = jnp.zeros((_L,), jnp.float32)
    init = (jnp.full((_L,), -3.0e38, jnp.float32), zeros, zeros, zeros, zeros)
    m_acc, np_acc, ll_acc, lc_acc, lct_acc = lax.fori_loop(
        0, nchunk, pass1, init)

    m_tot = jnp.max(m_acc)
    np_tot = jnp.sum(np_acc)
    ll_tot = jnp.sum(ll_acc)
    lc_tot = jnp.sum(lc_acc)
    lct_tot = jnp.sum(lct_acc)

    max_iou = jnp.where(np_tot > 0.0, m_tot, 2.0)
    thr = jnp.minimum(jnp.float32(0.5), max_iou)

    def pass2(i, acc):
        pl_acc, pc_acc, pnp_acc = acc
        o = i * _L
        iou = scr[pl.ds(0 * P + o, _L)]
        posf = scr[pl.ds(1 * P + o, _L)]
        lt0 = scr[pl.ds(2 * P + o, _L)]
        lt1 = scr[pl.ds(3 * P + o, _L)]
        ppf = posf * (iou >= thr).astype(jnp.float32)

        ll = xv[pl.ds(2 * P + o, _L)]
        lr = xv[pl.ds(3 * P + o, _L)]
        p0 = xv[pl.ds(6 * P + o, _L)]
        p1 = xv[pl.ds(7 * P + o, _L)]
        half_w = 0.5 * (ll + lr)
        plt0 = (lt0 - ll) / half_w
        plt1 = (lt1 - lr) / half_w
        d0 = jnp.abs(p0 - plt0)
        d1 = jnp.abs(p1 - plt1)
        sl0 = jnp.where(d0 < 1.0, 0.5 * d0 * d0, d0 - 0.5)
        sl1 = jnp.where(d1 < 1.0, 0.5 * d1 * d1, d1 - 0.5)

        q0 = xv[pl.ds(8 * P + o, _L)]
        q1 = xv[pl.ds(9 * P + o, _L)]
        fc = _focal_v(q0, q1, ppf == 0.0)

        pl_acc = pl_acc + (sl0 + sl1) * ppf
        pc_acc = pc_acc + fc
        pnp_acc = pnp_acc + ppf
        return pl_acc, pc_acc, pnp_acc

    pl_acc, pc_acc, pnp_acc = lax.fori_loop(
        0, nchunk, pass2, (zeros, zeros, zeros))

    plp_tot = jnp.sum(pl_acc)
    pc_tot = jnp.sum(pc_acc)
    pnp_tot = jnp.sum(pnp_acc)

    np_c = jnp.maximum(np_tot, 1.0)
    pnp_c = jnp.maximum(pnp_tot, 1.0)
    num = _lanes([ll_tot, lc_tot, lct_tot, plp_tot, pc_tot])
    den = jnp.where(lane < 3, np_c, pnp_c)
    ov[...] = num / den
    pltpu.sync_copy(ov, out_hbm.at[b])


@jax.jit
def kernel(loc_data, conf_data, prop_loc_data, prop_conf_data, center_data,
           priors, act_data, prop_act_data, targets):
    B, K, _ = loc_data.shape
    feat = jnp.stack([
        jnp.broadcast_to(priors[:, 0], (B, K)),
        jnp.broadcast_to(priors[:, 1], (B, K)),
        loc_data[:, :, 0], loc_data[:, :, 1],
        conf_data[:, :, 0], conf_data[:, :, 1],
        prop_loc_data[:, :, 0], prop_loc_data[:, :, 1],
        prop_conf_data[:, :, 0], prop_conf_data[:, :, 1],
        center_data[:, :, 0],
    ], axis=1)
    x = feat.reshape(B, _NROW * _PTILE)
    t = (jnp.broadcast_to(targets[:, :, :2, None], (B, _N_TGT, 2, _L))
         .reshape(B, _N_TGT * 2 * _L))

    mesh = plsc.VectorSubcoreMesh(core_axis_name="c", subcore_axis_name="s")
    run = functools.partial(
        pl.kernel,
        out_type=jax.ShapeDtypeStruct((B, _L), jnp.float32),
        mesh=mesh,
        compiler_params=pltpu.CompilerParams(needs_layout_passes=False),
        scratch_types=[
            pltpu.VMEM((_NROW * _PTILE,), jnp.float32),
            pltpu.VMEM((_N_TGT * 2 * _L,), jnp.float32),
            pltpu.VMEM((4 * _PTILE,), jnp.float32),
            pltpu.VMEM((_L,), jnp.float32),
        ],
    )(_sc_body)
    out = run(x, t)
    return jnp.mean(out[:, :5], axis=0)


# speedup vs baseline: 1.2774x; 1.2774x over previous
import functools

import jax
import jax.numpy as jnp
import numpy as np
from jax import lax
from jax.experimental import pallas as pl
from jax.experimental.pallas import tpu as pltpu
from jax.experimental.pallas import tpu_sc as plsc

_CLIP = 256.0
_EPS = float(np.finfo(np.float32).eps)
_LB = tuple(v / 256.0 for v in (0.0, 15.0, 30.0, 60.0, 96.0, 256.0))
_RB = tuple(v / 256.0 for v in (30.0, 60.0, 120.0, 240.0, 768.0, 768.0))
_N_TGT = 30
_L = 16
_PTILE = 4096
_NROW = 11


def _levels_to_bounds(lvlf):
    lb = jnp.full_like(lvlf, _LB[0])
    rb = jnp.full_like(lvlf, _RB[0])
    for i in range(1, 6):
        sel = lvlf > (i - 0.5)
        lb = jnp.where(sel, _LB[i], lb)
        rb = jnp.where(sel, _RB[i], rb)
    return lb, rb


def _sc_log(x):
    bits = lax.bitcast_convert_type(x, jnp.int32)
    e = jnp.bitwise_and(lax.shift_right_logical(bits, 23), 0xFF)
    ef = (e - 127).astype(jnp.float32)
    mbits = jnp.bitwise_or(jnp.bitwise_and(bits, 0x007FFFFF), 0x3F800000)
    m = lax.bitcast_convert_type(mbits, jnp.float32)
    big = m > 1.4142135381698608
    m = jnp.where(big, 0.5 * m, m)
    ef = jnp.where(big, ef + 1.0, ef)
    t = (m - 1.0) / (m + 1.0)
    t2 = t * t
    p = t * (2.0 + t2 * (0.6666666666666666
                         + t2 * (0.4 + t2 * 0.2857142857142857)))
    return ef * 0.6931471805599453 + p


def _iou_v(pl0, pl1, tl0, tl1):
    inter = jnp.minimum(pl0, tl0) + jnp.minimum(pl1, tl1)
    union = (tl0 + tl1) + (pl0 + pl1) - inter
    return inter / jnp.maximum(union, _EPS)


def _focal_v(a, b, is0):
    mx = jnp.maximum(a, b)
    za = jnp.exp(a - mx)
    zb = jnp.exp(b - mx)
    pt = jnp.where(is0, za, zb) / (za + zb)
    alpha_t = jnp.where(is0, 0.25, 0.75)
    omp = 1.0 - pt
    return -alpha_t * omp * omp * _sc_log(jnp.maximum(pt, _EPS))


def _lanes(vals):
    lane = jnp.arange(_L, dtype=jnp.int32)
    v = jnp.zeros((_L,), jnp.float32)
    for i, s in enumerate(vals):
        v = jnp.where(lane == i, s, v)
    return v


def _sc_body(x_hbm, t_hbm, out_hbm, ex_hbm, xv, tv, scr, exv, pxv, ov):
    ci = lax.axis_index("c")
    si = lax.axis_index("s")
    b = ci * 8 + si // 2
    half = si % 2
    wid = ci * 16 + si
    pltpu.sync_copy(x_hbm.at[b * 2 + half], xv)
    pltpu.sync_copy(t_hbm.at[b], tv)

    nchunk = _PTILE // _L
    P = _PTILE
    lane = jnp.arange(_L, dtype=jnp.int32)

    def pass1(i, acc):
        m_acc, np_acc, ll_acc, lc_acc, lct_acc = acc
        o = i * _L
        c = xv[pl.ds(0 * P + o, _L)]
        lvl = xv[pl.ds(1 * P + o, _L)]
        lb, rb = _levels_to_bounds(lvl)

        best = jnp.full((_L,), 2.0, jnp.float32)
        bs = jnp.zeros((_L,), jnp.float32)
        be = jnp.zeros((_L,), jnp.float32)
        for m in range(_N_TGT):
            s = tv[pl.ds(m * 2 * _L, _L)]
            e = tv[pl.ds(m * 2 * _L + _L, _L)]
            t1 = c - s
            t2 = e - c
            a = t1 + t2
            mn = jnp.minimum(t1, t2)
            mx = jnp.maximum(t1, t2)
            take = (mn >= 0.0) & (mx > lb) & (mx <= rb) & (a < best)
            best = jnp.where(take, a, best)
            bs = jnp.where(take, s, bs)
            be = jnp.where(take, e, be)

        found = best < 2.0
        posf = found.astype(jnp.float32)
        lt0 = (c - bs) * _CLIP
        lt1 = (be - c) * _CLIP
        ll = xv[pl.ds(2 * P + o, _L)]
        lr = xv[pl.ds(3 * P + o, _L)]
        iou = _iou_v(ll, lr, lt0, lt1)

        scr[pl.ds(0 * P + o, _L)] = iou
        scr[pl.ds(1 * P + o, _L)] = posf
        scr[pl.ds(2 * P + o, _L)] = lt0
        scr[pl.ds(3 * P + o, _L)] = lt1

        pred_area = ll + lr
        target_area = lt0 + lt1
        inter = jnp.minimum(ll, lt0) + jnp.minimum(lr, lt1)
        union = target_area + pred_area - inter
        ious = inter / jnp.maximum(union, _EPS)
        ac = jnp.maximum(ll, lt0) + jnp.maximum(lr, lt1)
        gious = ious - (ac - union) / jnp.maximum(ac, _EPS)

        p0 = xv[pl.ds(6 * P + o, _L)]
        p1 = xv[pl.ds(7 * P + o, _L)]
        ctr = xv[pl.ds(10 * P + o, _L)]
        half_w = 0.5 * pred_area
        cur0 = half_w * p0 + ll
        cur1 = half_w * p1 + lr
        ious_ct = jnp.maximum(_iou_v(cur0, cur1, lt0, lt1), 0.0)
        bce = (jnp.maximum(ctr, 0.0) - ctr * ious_ct
               + _sc_log(1.0 + jnp.exp(-jnp.abs(ctr))))

        g0 = xv[pl.ds(4 * P + o, _L)]
        g1 = xv[pl.ds(5 * P + o, _L)]
        fc = _focal_v(g0, g1, jnp.logical_not(found))

        m_acc = jnp.maximum(m_acc, jnp.where(found, iou, -3.0e38))
        np_acc = np_acc + posf
        ll_acc = ll_acc + (1.0 - gious) * posf
        lc_acc = lc_acc + fc
        lct_acc = lct_acc + bce * posf
        return m_acc, np_acc, ll_acc, lc_acc, lct_acc

    zeros = jnp.zeros((_L,), jnp.float32)
    init = (jnp.full((_L,), -3.0e38, jnp.float32), zeros, zeros, zeros, zeros)
    m_acc, np_acc, ll_acc, lc_acc, lct_acc = lax.fori_loop(
        0, nchunk, pass1, init)

    v1 = _lanes([jnp.max(m_acc), jnp.sum(np_acc), jnp.sum(ll_acc),
                 jnp.sum(lc_acc), jnp.sum(lct_acc)])
    exv[...] = v1
    pltpu.sync_copy(exv, ex_hbm.at[wid])
    plsc.subcore_barrier()
    pltpu.sync_copy(ex_hbm.at[wid ^ 1], pxv)
    p1v = pxv[...]
    comb = jnp.where(lane == 0, jnp.maximum(v1, p1v), v1 + p1v)

    def _at(vec, i):
        return jnp.sum(jnp.where(lane == i, vec, 0.0))

    m_tot = _at(comb, 0)
    np_tot = _at(comb, 1)
    ll_tot = _at(comb, 2)
    lc_tot = _at(comb, 3)
    lct_tot = _at(comb, 4)

    max_iou = jnp.where(np_tot > 0.0, m_tot, 2.0)
    thr = jnp.minimum(jnp.float32(0.5), max_iou)

    def pass2(i, acc):
        pl_acc, pc_acc, pnp_acc = acc
        o = i * _L
        iou = scr[pl.ds(0 * P + o, _L)]
        posf = scr[pl.ds(1 * P + o, _L)]
        lt0 = scr[pl.ds(2 * P + o, _L)]
        lt1 = scr[pl.ds(3 * P + o, _L)]
        ppf = posf * (iou >= thr).astype(jnp.float32)

        ll = xv[pl.ds(2 * P + o, _L)]
        lr = xv[pl.ds(3 * P + o, _L)]
        p0 = xv[pl.ds(6 * P + o, _L)]
        p1 = xv[pl.ds(7 * P + o, _L)]
        half_w = 0.5 * (ll + lr)
        plt0 = (lt0 - ll) / half_w
        plt1 = (lt1 - lr) / half_w
        d0 = jnp.abs(p0 - plt0)
        d1 = jnp.abs(p1 - plt1)
        sl0 = jnp.where(d0 < 1.0, 0.5 * d0 * d0, d0 - 0.5)
        sl1 = jnp.where(d1 < 1.0, 0.5 * d1 * d1, d1 - 0.5)

        q0 = xv[pl.ds(8 * P + o, _L)]
        q1 = xv[pl.ds(9 * P + o, _L)]
        fc = _focal_v(q0, q1, ppf == 0.0)

        pl_acc = pl_acc + (sl0 + sl1) * ppf
        pc_acc = pc_acc + fc
        pnp_acc = pnp_acc + ppf
        return pl_acc, pc_acc, pnp_acc

    pl_acc, pc_acc, pnp_acc = lax.fori_loop(
        0, nchunk, pass2, (zeros, zeros, zeros))

    v2 = _lanes([jnp.sum(pl_acc), jnp.sum(pc_acc), jnp.sum(pnp_acc)])
    exv[...] = v2
    pltpu.sync_copy(exv, ex_hbm.at[32 + wid])
    plsc.subcore_barrier()
    pltpu.sync_copy(ex_hbm.at[32 + (wid ^ 1)], pxv)
    comb2 = v2 + pxv[...]
    plp_tot = _at(comb2, 0)
    pc_tot = _at(comb2, 1)
    pnp_tot = _at(comb2, 2)

    @pl.when(half == 0)
    def _():
        np_c = jnp.maximum(np_tot, 1.0)
        pnp_c = jnp.maximum(pnp_tot, 1.0)
        num = _lanes([ll_tot, lc_tot, lct_tot, plp_tot, pc_tot])
        den = jnp.where(lane < 3, np_c, pnp_c)
        ov[...] = num / den
        pltpu.sync_copy(ov, out_hbm.at[b])


@jax.jit
def kernel(loc_data, conf_data, prop_loc_data, prop_conf_data, center_data,
           priors, act_data, prop_act_data, targets):
    B, K, _ = loc_data.shape
    feat = jnp.stack([
        jnp.broadcast_to(priors[:, 0], (B, K)),
        jnp.broadcast_to(priors[:, 1], (B, K)),
        loc_data[:, :, 0], loc_data[:, :, 1],
        conf_data[:, :, 0], conf_data[:, :, 1],
        prop_loc_data[:, :, 0], prop_loc_data[:, :, 1],
        prop_conf_data[:, :, 0], prop_conf_data[:, :, 1],
        center_data[:, :, 0],
    ], axis=1)
    x = (feat.reshape(B, _NROW, 2, _PTILE)
         .transpose(0, 2, 1, 3)
         .reshape(B * 2, _NROW * _PTILE))
    t = (jnp.broadcast_to(targets[:, :, :2, None], (B, _N_TGT, 2, _L))
         .reshape(B, _N_TGT * 2 * _L))

    mesh = plsc.VectorSubcoreMesh(core_axis_name="c", subcore_axis_name="s")
    run = functools.partial(
        pl.kernel,
        out_type=(jax.ShapeDtypeStruct((B, _L), jnp.float32),
                  jax.ShapeDtypeStruct((64, _L), jnp.float32)),
        mesh=mesh,
        compiler_params=pltpu.CompilerParams(needs_layout_passes=False),
        scratch_types=[
            pltpu.VMEM((_NROW * _PTILE,), jnp.float32),
            pltpu.VMEM((_N_TGT * 2 * _L,), jnp.float32),
            pltpu.VMEM((4 * _PTILE,), jnp.float32),
            pltpu.VMEM((_L,), jnp.float32),
            pltpu.VMEM((_L,), jnp.float32),
            pltpu.VMEM((_L,), jnp.float32),
        ],
    )(_sc_body)
    out, _ = run(x, t)
    return jnp.mean(out[:, :5], axis=0)


# speedup vs baseline: 1.3544x; 1.0602x over previous
import functools

import jax
import jax.numpy as jnp
import numpy as np
from jax import lax
from jax.experimental import pallas as pl
from jax.experimental.pallas import tpu as pltpu
from jax.experimental.pallas import tpu_sc as plsc

_CLIP = 256.0
_EPS = float(np.finfo(np.float32).eps)
_LB = tuple(v / 256.0 for v in (0.0, 15.0, 30.0, 60.0, 96.0, 256.0))
_RB = tuple(v / 256.0 for v in (30.0, 60.0, 120.0, 240.0, 768.0, 768.0))
_N_TGT = 30
_L = 16
_PTILE = 4096
_NROW = 11


def _levels_to_bounds(lvlf):
    lb = jnp.full_like(lvlf, _LB[0])
    rb = jnp.full_like(lvlf, _RB[0])
    for i in range(1, 6):
        sel = lvlf > (i - 0.5)
        lb = jnp.where(sel, _LB[i], lb)
        rb = jnp.where(sel, _RB[i], rb)
    return lb, rb


def _sc_log(x):
    bits = lax.bitcast_convert_type(x, jnp.int32)
    e = jnp.bitwise_and(lax.shift_right_logical(bits, 23), 0xFF)
    ef = (e - 127).astype(jnp.float32)
    mbits = jnp.bitwise_or(jnp.bitwise_and(bits, 0x007FFFFF), 0x3F800000)
    m = lax.bitcast_convert_type(mbits, jnp.float32)
    big = m > 1.4142135381698608
    m = jnp.where(big, 0.5 * m, m)
    ef = jnp.where(big, ef + 1.0, ef)
    t = (m - 1.0) / (m + 1.0)
    t2 = t * t
    p = t * (2.0 + t2 * (0.6666666666666666
                         + t2 * (0.4 + t2 * 0.2857142857142857)))
    return ef * 0.6931471805599453 + p


def _iou_v(pl0, pl1, tl0, tl1):
    inter = jnp.minimum(pl0, tl0) + jnp.minimum(pl1, tl1)
    union = (tl0 + tl1) + (pl0 + pl1) - inter
    return inter / jnp.maximum(union, _EPS)


def _focal_v(a, b, is0):
    mx = jnp.maximum(a, b)
    za = jnp.exp(a - mx)
    zb = jnp.exp(b - mx)
    pt = jnp.where(is0, za, zb) / (za + zb)
    alpha_t = jnp.where(is0, 0.25, 0.75)
    omp = 1.0 - pt
    return -alpha_t * omp * omp * _sc_log(jnp.maximum(pt, _EPS))


def _lanes(vals):
    lane = jnp.arange(_L, dtype=jnp.int32)
    v = jnp.zeros((_L,), jnp.float32)
    for i, s in enumerate(vals):
        v = jnp.where(lane == i, s, v)
    return v


def _sc_body(x_hbm, t_hbm, out_hbm, ex_hbm, xv, tv, scr, exv, pxv, ov):
    ci = lax.axis_index("c")
    si = lax.axis_index("s")
    b = ci * 8 + si // 2
    half = si % 2
    wid = ci * 16 + si
    pltpu.sync_copy(x_hbm.at[b * 2 + half], xv)
    pltpu.sync_copy(t_hbm.at[b], tv)

    nchunk = _PTILE // _L
    P = _PTILE
    lane = jnp.arange(_L, dtype=jnp.int32)

    def pass1(i, acc):
        m_acc, np_acc, ll_acc, lc_acc, lct_acc = acc
        o = i * _L
        c = xv[pl.ds(0 * P + o, _L)]
        lvl = xv[pl.ds(1 * P + o, _L)]
        lb, rb = _levels_to_bounds(lvl)

        best = jnp.full((_L,), 2.0, jnp.float32)
        bs = jnp.zeros((_L,), jnp.float32)
        be = jnp.zeros((_L,), jnp.float32)
        for m in range(_N_TGT):
            s = tv[pl.ds(m * 2 * _L, _L)]
            e = tv[pl.ds(m * 2 * _L + _L, _L)]
            t1 = c - s
            t2 = e - c
            a = t1 + t2
            mn = jnp.minimum(t1, t2)
            mx = jnp.maximum(t1, t2)
            take = (mn >= 0.0) & (mx > lb) & (mx <= rb) & (a < best)
            best = jnp.where(take, a, best)
            bs = jnp.where(take, s, bs)
            be = jnp.where(take, e, be)

        found = best < 2.0
        posf = found.astype(jnp.float32)
        lt0 = (c - bs) * _CLIP
        lt1 = (be - c) * _CLIP
        ll = xv[pl.ds(2 * P + o, _L)]
        lr = xv[pl.ds(3 * P + o, _L)]
        iou = _iou_v(ll, lr, lt0, lt1)

        scr[pl.ds(0 * P + o, _L)] = iou
        scr[pl.ds(1 * P + o, _L)] = posf
        scr[pl.ds(2 * P + o, _L)] = lt0
        scr[pl.ds(3 * P + o, _L)] = lt1

        pred_area = ll + lr
        target_area = lt0 + lt1
        inter = jnp.minimum(ll, lt0) + jnp.minimum(lr, lt1)
        union = target_area + pred_area - inter
        ious = inter / jnp.maximum(union, _EPS)
        ac = jnp.maximum(ll, lt0) + jnp.maximum(lr, lt1)
        gious = ious - (ac - union) / jnp.maximum(ac, _EPS)

        p0 = xv[pl.ds(6 * P + o, _L)]
        p1 = xv[pl.ds(7 * P + o, _L)]
        ctr = xv[pl.ds(10 * P + o, _L)]
        half_w = 0.5 * pred_area
        cur0 = half_w * p0 + ll
        cur1 = half_w * p1 + lr
        ious_ct = jnp.maximum(_iou_v(cur0, cur1, lt0, lt1), 0.0)
        bce = (jnp.maximum(ctr, 0.0) - ctr * ious_ct
               + _sc_log(1.0 + jnp.exp(-jnp.abs(ctr))))

        g0 = xv[pl.ds(4 * P + o, _L)]
        g1 = xv[pl.ds(5 * P + o, _L)]
        fc = _focal_v(g0, g1, jnp.logical_not(found))

        m_acc = jnp.maximum(m_acc, jnp.where(found, iou, -3.0e38))
        np_acc = np_acc + posf
        ll_acc = ll_acc + (1.0 - gious) * posf
        lc_acc = lc_acc + fc
        lct_acc = lct_acc + bce * posf
        return m_acc, np_acc, ll_acc, lc_acc, lct_acc

    zeros = jnp.zeros((_L,), jnp.float32)
    init = (jnp.full((_L,), -3.0e38, jnp.float32), zeros, zeros, zeros, zeros)
    m_acc, np_acc, ll_acc, lc_acc, lct_acc = lax.fori_loop(
        0, nchunk, pass1, init, unroll=2)

    v1 = _lanes([jnp.max(m_acc), jnp.sum(np_acc), jnp.sum(ll_acc),
                 jnp.sum(lc_acc), jnp.sum(lct_acc)])
    exv[...] = v1
    pltpu.sync_copy(exv, ex_hbm.at[wid])
    plsc.subcore_barrier()
    pltpu.sync_copy(ex_hbm.at[wid ^ 1], pxv)
    p1v = pxv[...]
    comb = jnp.where(lane == 0, jnp.maximum(v1, p1v), v1 + p1v)

    def _at(vec, i):
        return jnp.sum(jnp.where(lane == i, vec, 0.0))

    m_tot = _at(comb, 0)
    np_tot = _at(comb, 1)
    ll_tot = _at(comb, 2)
    lc_tot = _at(comb, 3)
    lct_tot = _at(comb, 4)

    max_iou = jnp.where(np_tot > 0.0, m_tot, 2.0)
    thr = jnp.minimum(jnp.float32(0.5), max_iou)

    def pass2(i, acc):
        pl_acc, pc_acc, pnp_acc = acc
        o = i * _L
        iou = scr[pl.ds(0 * P + o, _L)]
        posf = scr[pl.ds(1 * P + o, _L)]
        lt0 = scr[pl.ds(2 * P + o, _L)]
        lt1 = scr[pl.ds(3 * P + o, _L)]
        ppf = posf * (iou >= thr).astype(jnp.float32)

        ll = xv[pl.ds(2 * P + o, _L)]
        lr = xv[pl.ds(3 * P + o, _L)]
        p0 = xv[pl.ds(6 * P + o, _L)]
        p1 = xv[pl.ds(7 * P + o, _L)]
        half_w = 0.5 * (ll + lr)
        plt0 = (lt0 - ll) / half_w
        plt1 = (lt1 - lr) / half_w
        d0 = jnp.abs(p0 - plt0)
        d1 = jnp.abs(p1 - plt1)
        sl0 = jnp.where(d0 < 1.0, 0.5 * d0 * d0, d0 - 0.5)
        sl1 = jnp.where(d1 < 1.0, 0.5 * d1 * d1, d1 - 0.5)

        q0 = xv[pl.ds(8 * P + o, _L)]
        q1 = xv[pl.ds(9 * P + o, _L)]
        fc = _focal_v(q0, q1, ppf == 0.0)

        pl_acc = pl_acc + (sl0 + sl1) * ppf
        pc_acc = pc_acc + fc
        pnp_acc = pnp_acc + ppf
        return pl_acc, pc_acc, pnp_acc

    pl_acc, pc_acc, pnp_acc = lax.fori_loop(
        0, nchunk, pass2, (zeros, zeros, zeros), unroll=2)

    v2 = _lanes([jnp.sum(pl_acc), jnp.sum(pc_acc), jnp.sum(pnp_acc)])
    exv[...] = v2
    pltpu.sync_copy(exv, ex_hbm.at[32 + wid])
    plsc.subcore_barrier()
    pltpu.sync_copy(ex_hbm.at[32 + (wid ^ 1)], pxv)
    comb2 = v2 + pxv[...]
    plp_tot = _at(comb2, 0)
    pc_tot = _at(comb2, 1)
    pnp_tot = _at(comb2, 2)

    @pl.when(half == 0)
    def _():
        np_c = jnp.maximum(np_tot, 1.0)
        pnp_c = jnp.maximum(pnp_tot, 1.0)
        num = _lanes([ll_tot, lc_tot, lct_tot, plp_tot, pc_tot])
        den = jnp.where(lane < 3, np_c, pnp_c)
        ov[...] = num / den
        pltpu.sync_copy(ov, out_hbm.at[b])


@jax.jit
def kernel(loc_data, conf_data, prop_loc_data, prop_conf_data, center_data,
           priors, act_data, prop_act_data, targets):
    B, K, _ = loc_data.shape
    feat = jnp.stack([
        jnp.broadcast_to(priors[:, 0], (B, K)),
        jnp.broadcast_to(priors[:, 1], (B, K)),
        loc_data[:, :, 0], loc_data[:, :, 1],
        conf_data[:, :, 0], conf_data[:, :, 1],
        prop_loc_data[:, :, 0], prop_loc_data[:, :, 1],
        prop_conf_data[:, :, 0], prop_conf_data[:, :, 1],
        center_data[:, :, 0],
    ], axis=1)
    x = (feat.reshape(B, _NROW, 2, _PTILE)
         .transpose(0, 2, 1, 3)
         .reshape(B * 2, _NROW * _PTILE))
    t = (jnp.broadcast_to(targets[:, :, :2, None], (B, _N_TGT, 2, _L))
         .reshape(B, _N_TGT * 2 * _L))

    mesh = plsc.VectorSubcoreMesh(core_axis_name="c", subcore_axis_name="s")
    run = functools.partial(
        pl.kernel,
        out_type=(jax.ShapeDtypeStruct((B, _L), jnp.float32),
                  jax.ShapeDtypeStruct((64, _L), jnp.float32)),
        mesh=mesh,
        compiler_params=pltpu.CompilerParams(needs_layout_passes=False),
        scratch_types=[
            pltpu.VMEM((_NROW * _PTILE,), jnp.float32),
            pltpu.VMEM((_N_TGT * 2 * _L,), jnp.float32),
            pltpu.VMEM((4 * _PTILE,), jnp.float32),
            pltpu.VMEM((_L,), jnp.float32),
            pltpu.VMEM((_L,), jnp.float32),
            pltpu.VMEM((_L,), jnp.float32),
        ],
    )(_sc_body)
    out, _ = run(x, t)
    return jnp.mean(out[:, :5], axis=0)


# speedup vs baseline: 1.3848x; 1.0225x over previous
import functools

import jax
import jax.numpy as jnp
import numpy as np
from jax import lax
from jax.experimental import pallas as pl
from jax.experimental.pallas import tpu as pltpu
from jax.experimental.pallas import tpu_sc as plsc

_CLIP = 256.0
_EPS = float(np.finfo(np.float32).eps)
_LB = tuple(v / 256.0 for v in (0.0, 15.0, 30.0, 60.0, 96.0, 256.0))
_RB = tuple(v / 256.0 for v in (30.0, 60.0, 120.0, 240.0, 768.0, 768.0))
_N_TGT = 30
_L = 16
_PTILE = 4096
_NROW = 11


def _levels_to_bounds(lvlf):
    lb = jnp.full_like(lvlf, _LB[0])
    rb = jnp.full_like(lvlf, _RB[0])
    for i in range(1, 6):
        sel = lvlf > (i - 0.5)
        lb = jnp.where(sel, _LB[i], lb)
        rb = jnp.where(sel, _RB[i], rb)
    return lb, rb


def _sc_log(x):
    bits = lax.bitcast_convert_type(x, jnp.int32)
    e = jnp.bitwise_and(lax.shift_right_logical(bits, 23), 0xFF)
    ef = (e - 127).astype(jnp.float32)
    mbits = jnp.bitwise_or(jnp.bitwise_and(bits, 0x007FFFFF), 0x3F800000)
    m = lax.bitcast_convert_type(mbits, jnp.float32)
    big = m > 1.4142135381698608
    m = jnp.where(big, 0.5 * m, m)
    ef = jnp.where(big, ef + 1.0, ef)
    t = (m - 1.0) / (m + 1.0)
    t2 = t * t
    p = t * (2.0 + t2 * (0.6666666666666666
                         + t2 * (0.4 + t2 * 0.2857142857142857)))
    return ef * 0.6931471805599453 + p


def _iou_v(pl0, pl1, tl0, tl1):
    inter = jnp.minimum(pl0, tl0) + jnp.minimum(pl1, tl1)
    union = (tl0 + tl1) + (pl0 + pl1) - inter
    return inter / jnp.maximum(union, _EPS)


def _focal_v(a, b, is0):
    mx = jnp.maximum(a, b)
    za = jnp.exp(a - mx)
    zb = jnp.exp(b - mx)
    pt = jnp.where(is0, za, zb) / (za + zb)
    alpha_t = jnp.where(is0, 0.25, 0.75)
    omp = 1.0 - pt
    return -alpha_t * omp * omp * _sc_log(jnp.maximum(pt, _EPS))


def _lanes(vals):
    lane = jnp.arange(_L, dtype=jnp.int32)
    v = jnp.zeros((_L,), jnp.float32)
    for i, s in enumerate(vals):
        v = jnp.where(lane == i, s, v)
    return v


def _sc_body(x_hbm, t_hbm, out_hbm, ex_hbm, xv, tv, scr, exv, pxv, ov):
    ci = lax.axis_index("c")
    si = lax.axis_index("s")
    b = ci * 8 + si // 2
    half = si % 2
    wid = ci * 16 + si
    pltpu.sync_copy(x_hbm.at[b * 2 + half], xv)
    pltpu.sync_copy(t_hbm.at[b], tv)

    nchunk = _PTILE // _L
    P = _PTILE
    lane = jnp.arange(_L, dtype=jnp.int32)

    def pass1(i, acc):
        m_acc, np_acc, ll_acc, lc_acc, lct_acc = acc
        o = i * _L
        c = xv[pl.ds(0 * P + o, _L)]
        lvl = xv[pl.ds(1 * P + o, _L)]
        lb, rb = _levels_to_bounds(lvl)

        best = jnp.full((_L,), 2.0, jnp.float32)
        bs = jnp.zeros((_L,), jnp.float32)
        be = jnp.zeros((_L,), jnp.float32)
        for m in range(_N_TGT):
            s = tv[pl.ds(m * 2 * _L, _L)]
            e = tv[pl.ds(m * 2 * _L + _L, _L)]
            t1 = c - s
            t2 = e - c
            a = t1 + t2
            mn = jnp.minimum(t1, t2)
            mx = jnp.maximum(t1, t2)
            take = (mn >= 0.0) & (mx > lb) & (mx <= rb) & (a < best)
            best = jnp.where(take, a, best)
            bs = jnp.where(take, s, bs)
            be = jnp.where(take, e, be)

        found = best < 2.0
        posf = found.astype(jnp.float32)
        lt0 = (c - bs) * _CLIP
        lt1 = (be - c) * _CLIP
        ll = xv[pl.ds(2 * P + o, _L)]
        lr = xv[pl.ds(3 * P + o, _L)]
        iou = _iou_v(ll, lr, lt0, lt1)

        scr[pl.ds(0 * P + o, _L)] = iou
        scr[pl.ds(1 * P + o, _L)] = posf
        scr[pl.ds(2 * P + o, _L)] = lt0
        scr[pl.ds(3 * P + o, _L)] = lt1

        pred_area = ll + lr
        target_area = lt0 + lt1
        inter = jnp.minimum(ll, lt0) + jnp.minimum(lr, lt1)
        union = target_area + pred_area - inter
        ious = inter / jnp.maximum(union, _EPS)
        ac = jnp.maximum(ll, lt0) + jnp.maximum(lr, lt1)
        gious = ious - (ac - union) / jnp.maximum(ac, _EPS)

        p0 = xv[pl.ds(6 * P + o, _L)]
        p1 = xv[pl.ds(7 * P + o, _L)]
        ctr = xv[pl.ds(10 * P + o, _L)]
        half_w = 0.5 * pred_area
        cur0 = half_w * p0 + ll
        cur1 = half_w * p1 + lr
        ious_ct = jnp.maximum(_iou_v(cur0, cur1, lt0, lt1), 0.0)
        bce = (jnp.maximum(ctr, 0.0) - ctr * ious_ct
               + _sc_log(1.0 + jnp.exp(-jnp.abs(ctr))))

        g0 = xv[pl.ds(4 * P + o, _L)]
        g1 = xv[pl.ds(5 * P + o, _L)]
        fc = _focal_v(g0, g1, jnp.logical_not(found))

        m_acc = jnp.maximum(m_acc, jnp.where(found, iou, -3.0e38))
        np_acc = np_acc + posf
        ll_acc = ll_acc + (1.0 - gious) * posf
        lc_acc = lc_acc + fc
        lct_acc = lct_acc + bce * posf
        return m_acc, np_acc, ll_acc, lc_acc, lct_acc

    zeros = jnp.zeros((_L,), jnp.float32)
    init = (jnp.full((_L,), -3.0e38, jnp.float32), zeros, zeros, zeros, zeros)
    m_acc, np_acc, ll_acc, lc_acc, lct_acc = lax.fori_loop(
        0, nchunk, pass1, init, unroll=4)

    v1 = _lanes([jnp.max(m_acc), jnp.sum(np_acc), jnp.sum(ll_acc),
                 jnp.sum(lc_acc), jnp.sum(lct_acc)])
    exv[...] = v1
    pltpu.sync_copy(exv, ex_hbm.at[wid])
    plsc.subcore_barrier()
    pltpu.sync_copy(ex_hbm.at[wid ^ 1], pxv)
    p1v = pxv[...]
    comb = jnp.where(lane == 0, jnp.maximum(v1, p1v), v1 + p1v)

    def _at(vec, i):
        return jnp.sum(jnp.where(lane == i, vec, 0.0))

    m_tot = _at(comb, 0)
    np_tot = _at(comb, 1)
    ll_tot = _at(comb, 2)
    lc_tot = _at(comb, 3)
    lct_tot = _at(comb, 4)

    max_iou = jnp.where(np_tot > 0.0, m_tot, 2.0)
    thr = jnp.minimum(jnp.float32(0.5), max_iou)

    def pass2(i, acc):
        pl_acc, pc_acc, pnp_acc = acc
        o = i * _L
        iou = scr[pl.ds(0 * P + o, _L)]
        posf = scr[pl.ds(1 * P + o, _L)]
        lt0 = scr[pl.ds(2 * P + o, _L)]
        lt1 = scr[pl.ds(3 * P + o, _L)]
        ppf = posf * (iou >= thr).astype(jnp.float32)

        ll = xv[pl.ds(2 * P + o, _L)]
        lr = xv[pl.ds(3 * P + o, _L)]
        p0 = xv[pl.ds(6 * P + o, _L)]
        p1 = xv[pl.ds(7 * P + o, _L)]
        half_w = 0.5 * (ll + lr)
        plt0 = (lt0 - ll) / half_w
        plt1 = (lt1 - lr) / half_w
        d0 = jnp.abs(p0 - plt0)
        d1 = jnp.abs(p1 - plt1)
        sl0 = jnp.where(d0 < 1.0, 0.5 * d0 * d0, d0 - 0.5)
        sl1 = jnp.where(d1 < 1.0, 0.5 * d1 * d1, d1 - 0.5)

        q0 = xv[pl.ds(8 * P + o, _L)]
        q1 = xv[pl.ds(9 * P + o, _L)]
        fc = _focal_v(q0, q1, ppf == 0.0)

        pl_acc = pl_acc + (sl0 + sl1) * ppf
        pc_acc = pc_acc + fc
        pnp_acc = pnp_acc + ppf
        return pl_acc, pc_acc, pnp_acc

    pl_acc, pc_acc, pnp_acc = lax.fori_loop(
        0, nchunk, pass2, (zeros, zeros, zeros), unroll=2)

    v2 = _lanes([jnp.sum(pl_acc), jnp.sum(pc_acc), jnp.sum(pnp_acc)])
    exv[...] = v2
    pltpu.sync_copy(exv, ex_hbm.at[32 + wid])
    plsc.subcore_barrier()
    pltpu.sync_copy(ex_hbm.at[32 + (wid ^ 1)], pxv)
    comb2 = v2 + pxv[...]
    plp_tot = _at(comb2, 0)
    pc_tot = _at(comb2, 1)
    pnp_tot = _at(comb2, 2)

    @pl.when(half == 0)
    def _():
        np_c = jnp.maximum(np_tot, 1.0)
        pnp_c = jnp.maximum(pnp_tot, 1.0)
        num = _lanes([ll_tot, lc_tot, lct_tot, plp_tot, pc_tot])
        den = jnp.where(lane < 3, np_c, pnp_c)
        ov[...] = num / den
        pltpu.sync_copy(ov, out_hbm.at[b])


@jax.jit
def kernel(loc_data, conf_data, prop_loc_data, prop_conf_data, center_data,
           priors, act_data, prop_act_data, targets):
    B, K, _ = loc_data.shape
    feat = jnp.stack([
        jnp.broadcast_to(priors[:, 0], (B, K)),
        jnp.broadcast_to(priors[:, 1], (B, K)),
        loc_data[:, :, 0], loc_data[:, :, 1],
        conf_data[:, :, 0], conf_data[:, :, 1],
        prop_loc_data[:, :, 0], prop_loc_data[:, :, 1],
        prop_conf_data[:, :, 0], prop_conf_data[:, :, 1],
        center_data[:, :, 0],
    ], axis=1)
    x = (feat.reshape(B, _NROW, 2, _PTILE)
         .transpose(0, 2, 1, 3)
         .reshape(B * 2, _NROW * _PTILE))
    t = (jnp.broadcast_to(targets[:, :, :2, None], (B, _N_TGT, 2, _L))
         .reshape(B, _N_TGT * 2 * _L))

    mesh = plsc.VectorSubcoreMesh(core_axis_name="c", subcore_axis_name="s")
    run = functools.partial(
        pl.kernel,
        out_type=(jax.ShapeDtypeStruct((B, _L), jnp.float32),
                  jax.ShapeDtypeStruct((64, _L), jnp.float32)),
        mesh=mesh,
        compiler_params=pltpu.CompilerParams(needs_layout_passes=False),
        scratch_types=[
            pltpu.VMEM((_NROW * _PTILE,), jnp.float32),
            pltpu.VMEM((_N_TGT * 2 * _L,), jnp.float32),
            pltpu.VMEM((4 * _PTILE,), jnp.float32),
            pltpu.VMEM((_L,), jnp.float32),
            pltpu.VMEM((_L,), jnp.float32),
            pltpu.VMEM((_L,), jnp.float32),
        ],
    )(_sc_body)
    out, _ = run(x, t)
    return jnp.mean(out[:, :5], axis=0)
